# baseline (speedup 1.0000x reference)
import jax
import jax.numpy as jnp
from jax import lax
from jax.experimental import pallas as pl
from jax.experimental.pallas import tpu as pltpu
from jax.experimental.pallas import tpu_sc as plsc

_N = 10000
_E = 160000
_HID = 512
_NCHUNK = 4
_CW = 128
_NSUB = 16
_B = 128
_EPT = 10240
_EPAD = _EPT * _NSUB
_NR = 10240
_RPT = _NR // _NSUB
_MB = 1000


def _sc_deg_body(dst_hbm, ones_hbm, zeros_hbm, deg_hbm, dst_v, dst_b, ones_v,
                 accd, sem):
    core = lax.axis_index("c")
    sid = lax.axis_index("s")

    @pl.when(core == 0)
    def _():
        pltpu.sync_copy(dst_hbm.at[pl.ds(sid * _EPT, _EPT)], dst_v)
        pltpu.sync_copy(ones_hbm, ones_v)
        pltpu.sync_copy(zeros_hbm.at[pl.ds(sid * _RPT, _RPT)],
                        accd.at[pl.ds(sid * _RPT, _RPT)])
        plsc.subcore_barrier()

        def block(b, carry):
            for j in range(_B // 16):
                dst_b[pl.ds(j * 16, 16)] = dst_v[pl.ds(b * _B + j * 16, 16)]
            pltpu.sync_copy(ones_v, accd.at[dst_b], add=True)
            return carry

        lax.fori_loop(0, _EPT // _B, block, 0)
        plsc.subcore_barrier()
        pltpu.sync_copy(accd.at[pl.ds(sid * _RPT, _RPT)],
                        deg_hbm.at[pl.ds(sid * _RPT, _RPT)])


def _sc_deg(dst_p, ones_rows, zeros_rows):
    return pl.kernel(
        _sc_deg_body,
        out_type=jax.ShapeDtypeStruct((_NR, _CW), jnp.float32),
        mesh=plsc.VectorSubcoreMesh(core_axis_name="c", subcore_axis_name="s"),
        scratch_types=[
            pltpu.VMEM((_EPT,), jnp.int32),
            pltpu.VMEM((_B,), jnp.int32),
            pltpu.VMEM((_B, _CW), jnp.float32),
            pltpu.VMEM_SHARED((_NR, _CW), jnp.float32),
            pltpu.SemaphoreType.DMA,
        ],
    )(dst_p, ones_rows, zeros_rows)


_NBUF = 4
_BB = 64
_NB = _EPT // _BB


def _sc_agg_body(xs_hbm, words_hbm, out_hbm, words_v,
                 idx0, idx1, idx2, idx3, dd0, dd1, dd2, dd3,
                 r0, r1, r2, r3, acc,
                 g0, g1, g2, g3, s0, s1, s2, s3):
    core = lax.axis_index("c")
    sid = lax.axis_index("s")
    idx = (idx0, idx1, idx2, idx3)
    dd = (dd0, dd1, dd2, dd3)
    rows = (r0, r1, r2, r3)
    gsem = (g0, g1, g2, g3)
    ssem = (s0, s1, s2, s3)
    pltpu.sync_copy(words_hbm.at[pl.ds(sid * _EPT, _EPT)], words_v)

    for cj in range(2):
        chunk = core * 2 + cj
        pltpu.sync_copy(xs_hbm.at[pl.ds(chunk * _NR + sid * _RPT, _RPT)],
                        acc.at[pl.ds(sid * _RPT, _RPT)])
        plsc.subcore_barrier()

        def fill(b, k):
            for j in range(_BB // 16):
                sl = pl.ds(j * 16, 16)
                w = words_v[pl.ds(b * _BB + j * 16, 16)]
                idx[k][sl] = (w & 0xFFFF) + chunk * _NR
                dd[k][sl] = lax.shift_right_logical(w, 16)

        for k in range(2):
            fill(k, k)
            pltpu.async_copy(xs_hbm.at[idx[k]], rows[k], gsem[k])

        def quad(p, carry):
            for q in range(_NBUF):
                b = p * _NBUF + q
                m = q
                mt = (q + 2) % _NBUF

                @pl.when(b + 2 < _NB)
                def _():
                    @pl.when(b >= 2)
                    def _():
                        pltpu.make_async_copy(
                            rows[mt], acc.at[dd[mt]], ssem[mt]).wait()
                    fill(b + 2, mt)
                    pltpu.async_copy(xs_hbm.at[idx[mt]], rows[mt], gsem[mt])

                pltpu.make_async_copy(
                    xs_hbm.at[idx[m]], rows[m], gsem[m]).wait()
                pltpu.async_copy(rows[m], acc.at[dd[m]], ssem[m], add=True)
            return carry

        lax.fori_loop(0, _NB // _NBUF, quad, 0)
        for b in range(_NB - _NBUF, _NB):
            m = b % _NBUF
            pltpu.make_async_copy(rows[m], acc.at[dd[m]], ssem[m]).wait()
        plsc.subcore_barrier()
        pltpu.sync_copy(acc.at[pl.ds(sid * _RPT, _RPT)],
                        out_hbm.at[chunk, pl.ds(sid * _RPT, _RPT)])


def _sc_agg(xs_flat, words_p):
    return pl.kernel(
        _sc_agg_body,
        out_type=jax.ShapeDtypeStruct((_NCHUNK, _NR, _CW), jnp.float32),
        mesh=plsc.VectorSubcoreMesh(core_axis_name="c", subcore_axis_name="s"),
        scratch_types=[
            pltpu.VMEM((_EPT,), jnp.int32),
            *[pltpu.VMEM((_BB,), jnp.int32) for _ in range(2 * _NBUF)],
            *[pltpu.VMEM((_BB, _CW), jnp.float32) for _ in range(_NBUF)],
            pltpu.VMEM_SHARED((_NR, _CW), jnp.float32),
            *[pltpu.SemaphoreType.DMA for _ in range(2 * _NBUF)],
        ],
    )(xs_flat, words_p)


def _dinv_body(deg_ref, o_ref):
    o_ref[...] = lax.rsqrt(deg_ref[:, 0:1] + 1.0)


def _dinv(deg):
    return pl.pallas_call(
        _dinv_body,
        grid=(_N // 2000,),
        in_specs=[pl.BlockSpec((2000, _CW), lambda i: (i, 0))],
        out_specs=pl.BlockSpec((2000, 1), lambda i: (i, 0)),
        out_shape=jax.ShapeDtypeStruct((_N, 1), jnp.float32),
    )(deg)


def _mmb_body(x_ref, w_ref, b_ref, o_ref):
    o_ref[...] = jnp.dot(x_ref[...], w_ref[...],
                         preferred_element_type=jnp.float32) + b_ref[...]


def _matmul_bias(x, w, b):
    k, n = w.shape
    return pl.pallas_call(
        _mmb_body,
        grid=(_N // _MB,),
        in_specs=[
            pl.BlockSpec((_MB, k), lambda i: (i, 0)),
            pl.BlockSpec((k, n), lambda i: (0, 0)),
            pl.BlockSpec((1, n), lambda i: (0, 0)),
        ],
        out_specs=pl.BlockSpec((_MB, n), lambda i: (i, 0)),
        out_shape=jax.ShapeDtypeStruct((_N, n), jnp.float32),
    )(x, w, b.reshape(1, n))


def _k1_body(h_ref, w_ref, dinv_ref, o_ref):
    y = jnp.dot(h_ref[...], w_ref[...],
                preferred_element_type=jnp.float32) * dinv_ref[...]
    for c in range(_NCHUNK):
        o_ref[c] = y[:, c * _CW:(c + 1) * _CW]


def _k1(h, wc, dinv):
    return pl.pallas_call(
        _k1_body,
        grid=(_N // _MB,),
        in_specs=[
            pl.BlockSpec((_MB, _HID), lambda i: (i, 0)),
            pl.BlockSpec((_HID, _HID), lambda i: (0, 0)),
            pl.BlockSpec((_MB, 1), lambda i: (i, 0)),
        ],
        out_specs=pl.BlockSpec((_NCHUNK, _MB, _CW), lambda i: (0, i, 0)),
        out_shape=jax.ShapeDtypeStruct((_NCHUNK, _NR, _CW), jnp.float32),
    )(h, wc, dinv)


def _k2_body(agg_ref, res_ref, dinv_ref, bc_ref, g_ref, be_ref, o_ref):
    agg = jnp.concatenate([agg_ref[c] for c in range(_NCHUNK)], axis=1)
    y = dinv_ref[...] * agg + bc_ref[...] + res_ref[...]
    mu = jnp.mean(y, axis=1, keepdims=True)
    yc = y - mu
    var = jnp.mean(yc * yc, axis=1, keepdims=True)
    ln = yc * lax.rsqrt(var + 1e-5) * g_ref[...] + be_ref[...]
    o_ref[...] = jnp.maximum(ln, 0.0)


def _k2(agg, res, dinv, bc, g, be):
    vec = lambda v: v.reshape(1, _HID)
    return pl.pallas_call(
        _k2_body,
        grid=(_N // _MB,),
        in_specs=[
            pl.BlockSpec((_NCHUNK, _MB, _CW), lambda i: (0, i, 0)),
            pl.BlockSpec((_MB, _HID), lambda i: (i, 0)),
            pl.BlockSpec((_MB, 1), lambda i: (i, 0)),
            pl.BlockSpec((1, _HID), lambda i: (0, 0)),
            pl.BlockSpec((1, _HID), lambda i: (0, 0)),
            pl.BlockSpec((1, _HID), lambda i: (0, 0)),
        ],
        out_specs=pl.BlockSpec((_MB, _HID), lambda i: (i, 0)),
        out_shape=jax.ShapeDtypeStruct((_N, _HID), jnp.float32),
    )(agg, res, dinv, vec(bc), vec(g), vec(be))


def kernel(x, edge_index, params):
    src = edge_index[0].astype(jnp.int32)
    dst = edge_index[1].astype(jnp.int32)
    npad = _EPAD - _E
    src_p = jnp.concatenate([src, jnp.zeros((npad,), jnp.int32)])
    dst_p = jnp.concatenate([dst, jnp.full((npad,), _N, jnp.int32)])

    words_p = jnp.bitwise_or(src_p, jnp.left_shift(dst_p, 16))

    ones_rows = jnp.ones((_B, _CW), jnp.float32)
    zeros_rows = jnp.zeros((_NR, _CW), jnp.float32)
    deg = _sc_deg(dst_p, ones_rows, zeros_rows)
    dinv = _dinv(deg)

    h = _matmul_bias(x, params['W_in'], params['b_in'])
    for l in range(4):
        xs = _k1(h, params[f'Wc{l}'], dinv)
        res = _matmul_bias(h, params[f'Wr{l}'], params[f'br{l}'])
        agg = _sc_agg(xs.reshape(_NCHUNK * _NR, _CW), words_p)
        h = _k2(agg, res, dinv,
                params[f'bc{l}'], params[f'g{l}'], params[f'be{l}'])
    return _matmul_bias(h, params['W_out'], params['b_out'])


# speedup vs baseline: 1.2865x; 1.2865x over previous
import jax
import jax.numpy as jnp
from jax import lax
from jax.experimental import pallas as pl
from jax.experimental.pallas import tpu as pltpu
from jax.experimental.pallas import tpu_sc as plsc

_N = 10000
_E = 160000
_HID = 512
_NCHUNK = 4
_CW = 128
_NSUB = 16
_B = 128
_EPT = 10240
_EPAD = _EPT * _NSUB
_NR = 10240
_RPT = _NR // _NSUB
_MB = 1000


def _sc_deg_body(dst_hbm, ones_hbm, zeros_hbm, deg_hbm, dst_v, dst_b, ones_v,
                 accd, sem):
    core = lax.axis_index("c")
    sid = lax.axis_index("s")

    @pl.when(core == 0)
    def _():
        pltpu.sync_copy(dst_hbm.at[pl.ds(sid * _EPT, _EPT)], dst_v)
        pltpu.sync_copy(ones_hbm, ones_v)
        pltpu.sync_copy(zeros_hbm.at[pl.ds(sid * _RPT, _RPT)],
                        accd.at[pl.ds(sid * _RPT, _RPT)])
        plsc.subcore_barrier()

        def block(b, carry):
            for j in range(_B // 16):
                dst_b[pl.ds(j * 16, 16)] = dst_v[pl.ds(b * _B + j * 16, 16)]
            pltpu.sync_copy(ones_v, accd.at[dst_b], add=True)
            return carry

        lax.fori_loop(0, _EPT // _B, block, 0)
        plsc.subcore_barrier()
        pltpu.sync_copy(accd.at[pl.ds(sid * _RPT, _RPT)],
                        deg_hbm.at[pl.ds(sid * _RPT, _RPT)])


def _sc_deg(dst_p, ones_rows, zeros_rows):
    return pl.kernel(
        _sc_deg_body,
        out_type=jax.ShapeDtypeStruct((_NR, _CW), jnp.float32),
        mesh=plsc.VectorSubcoreMesh(core_axis_name="c", subcore_axis_name="s"),
        scratch_types=[
            pltpu.VMEM((_EPT,), jnp.int32),
            pltpu.VMEM((_B,), jnp.int32),
            pltpu.VMEM((_B, _CW), jnp.float32),
            pltpu.VMEM_SHARED((_NR, _CW), jnp.float32),
            pltpu.SemaphoreType.DMA,
        ],
    )(dst_p, ones_rows, zeros_rows)


_NBUF = 2
_BB = 64
_NB = _EPT // _BB


def _sc_agg_body(xs_hbm, xsb_hbm, words_hbm, out_hbm, words_v,
                 idx0, idx1, dd0, dd1, rb0, rb1, rf0, rf1, acc,
                 g0, g1, s0, s1):
    core = lax.axis_index("c")
    sid = lax.axis_index("s")
    idx = (idx0, idx1)
    dd = (dd0, dd1)
    rows_b = (rb0, rb1)
    rows_f = (rf0, rf1)
    gsem = (g0, g1)
    ssem = (s0, s1)
    pltpu.sync_copy(words_hbm.at[pl.ds(sid * _EPT, _EPT)], words_v)

    for cj in range(2):
        chunk = core * 2 + cj
        pltpu.sync_copy(xs_hbm.at[pl.ds(chunk * _NR + sid * _RPT, _RPT)],
                        acc.at[pl.ds(sid * _RPT, _RPT)])
        plsc.subcore_barrier()

        def fill(b, k):
            for j in range(_BB // 16):
                sl = pl.ds(j * 16, 16)
                w = words_v[pl.ds(b * _BB + j * 16, 16)]
                idx[k][sl] = (w & 0xFFFF) + chunk * _NR
                dd[k][sl] = lax.shift_right_logical(w, 16)

        def convert(k):
            for r in range(_BB):
                for g in range(_CW // 32):
                    w = rows_b[k][r, pl.ds(g * 16, 16)]
                    fe = lax.bitcast_convert_type(
                        lax.shift_left(w, 16), jnp.float32)
                    fo = lax.bitcast_convert_type(
                        w & jnp.int32(-65536), jnp.float32)
                    rows_f[k][r, pl.ds(g * 32, 16)] = fe
                    rows_f[k][r, pl.ds(g * 32 + 16, 16)] = fo

        fill(0, 0)
        pltpu.async_copy(xsb_hbm.at[idx0], rb0, g0)

        def pair(p, carry):
            for q in range(_NBUF):
                b = p * _NBUF + q
                m = q
                mn = 1 - q

                @pl.when(b + 1 < _NB)
                def _():
                    @pl.when(b >= 1)
                    def _():
                        pltpu.make_async_copy(
                            rows_f[mn], acc.at[dd[mn]], ssem[mn]).wait()
                    fill(b + 1, mn)
                    pltpu.async_copy(xsb_hbm.at[idx[mn]], rows_b[mn], gsem[mn])

                pltpu.make_async_copy(
                    xsb_hbm.at[idx[m]], rows_b[m], gsem[m]).wait()
                convert(m)
                pltpu.async_copy(rows_f[m], acc.at[dd[m]], ssem[m], add=True)
            return carry

        lax.fori_loop(0, _NB // _NBUF, pair, 0)
        for b in range(_NB - _NBUF, _NB):
            m = b % _NBUF
            pltpu.make_async_copy(rows_f[m], acc.at[dd[m]], ssem[m]).wait()
        plsc.subcore_barrier()
        pltpu.sync_copy(acc.at[pl.ds(sid * _RPT, _RPT)],
                        out_hbm.at[chunk, pl.ds(sid * _RPT, _RPT)])


def _sc_agg(xs_flat, xsb_flat, words_p):
    return pl.kernel(
        _sc_agg_body,
        out_type=jax.ShapeDtypeStruct((_NCHUNK, _NR, _CW), jnp.float32),
        mesh=plsc.VectorSubcoreMesh(core_axis_name="c", subcore_axis_name="s"),
        compiler_params=pltpu.CompilerParams(use_tc_tiling_on_sc=False),
        scratch_types=[
            pltpu.VMEM((_EPT,), jnp.int32),
            *[pltpu.VMEM((_BB,), jnp.int32) for _ in range(2 * _NBUF)],
            *[pltpu.VMEM((_BB, _CW // 2), jnp.int32) for _ in range(_NBUF)],
            *[pltpu.VMEM((_BB, _CW), jnp.float32) for _ in range(_NBUF)],
            pltpu.VMEM_SHARED((_NR, _CW), jnp.float32),
            *[pltpu.SemaphoreType.DMA for _ in range(2 * _NBUF)],
        ],
    )(xs_flat, xsb_flat, words_p)


def _dinv_body(deg_ref, o_ref):
    o_ref[...] = lax.rsqrt(deg_ref[:, 0:1] + 1.0)


def _dinv(deg):
    return pl.pallas_call(
        _dinv_body,
        grid=(_N // 2000,),
        in_specs=[pl.BlockSpec((2000, _CW), lambda i: (i, 0))],
        out_specs=pl.BlockSpec((2000, 1), lambda i: (i, 0)),
        out_shape=jax.ShapeDtypeStruct((_N, 1), jnp.float32),
    )(deg)


def _mmb_body(x_ref, w_ref, b_ref, o_ref):
    o_ref[...] = jnp.dot(x_ref[...], w_ref[...],
                         preferred_element_type=jnp.float32) + b_ref[...]


def _matmul_bias(x, w, b):
    k, n = w.shape
    return pl.pallas_call(
        _mmb_body,
        grid=(_N // _MB,),
        in_specs=[
            pl.BlockSpec((_MB, k), lambda i: (i, 0)),
            pl.BlockSpec((k, n), lambda i: (0, 0)),
            pl.BlockSpec((1, n), lambda i: (0, 0)),
        ],
        out_specs=pl.BlockSpec((_MB, n), lambda i: (i, 0)),
        out_shape=jax.ShapeDtypeStruct((_N, n), jnp.float32),
    )(x, w, b.reshape(1, n))


def _bf16_bits(u):
    c16 = jnp.uint32(16)
    return lax.shift_right_logical(
        u + jnp.uint32(0x7FFF) + (lax.shift_right_logical(u, c16)
                                  & jnp.uint32(1)), c16)


def _k1_body(h_ref, w_ref, dinv_ref, o_ref, ob_ref):
    y = jnp.dot(h_ref[...], w_ref[...],
                preferred_element_type=jnp.float32) * dinv_ref[...]
    a = jnp.concatenate([y[:, 32 * g:32 * g + 16] for g in range(_HID // 32)],
                        axis=1)
    b = jnp.concatenate([y[:, 32 * g + 16:32 * g + 32]
                         for g in range(_HID // 32)], axis=1)
    ra = _bf16_bits(lax.bitcast_convert_type(a, jnp.uint32))
    rb = _bf16_bits(lax.bitcast_convert_type(b, jnp.uint32))
    w = lax.bitcast_convert_type((rb << jnp.uint32(16)) | ra, jnp.int32)
    for c in range(_NCHUNK):
        o_ref[c] = y[:, c * _CW:(c + 1) * _CW]
        ob_ref[c] = w[:, (_CW // 2) * c:(_CW // 2) * (c + 1)]


def _k1(h, wc, dinv):
    return pl.pallas_call(
        _k1_body,
        grid=(_N // _MB,),
        in_specs=[
            pl.BlockSpec((_MB, _HID), lambda i: (i, 0)),
            pl.BlockSpec((_HID, _HID), lambda i: (0, 0)),
            pl.BlockSpec((_MB, 1), lambda i: (i, 0)),
        ],
        out_specs=[
            pl.BlockSpec((_NCHUNK, _MB, _CW), lambda i: (0, i, 0)),
            pl.BlockSpec((_NCHUNK, _MB, _CW // 2), lambda i: (0, i, 0)),
        ],
        out_shape=[
            jax.ShapeDtypeStruct((_NCHUNK, _NR, _CW), jnp.float32),
            jax.ShapeDtypeStruct((_NCHUNK, _NR, _CW // 2), jnp.int32),
        ],
    )(h, wc, dinv)


def _k2_body(agg_ref, h_ref, wr_ref, dinv_ref, bc_ref, br_ref, g_ref, be_ref,
             o_ref):
    agg = jnp.concatenate([agg_ref[c] for c in range(_NCHUNK)], axis=1)
    y = (dinv_ref[...] * agg + bc_ref[...]
         + jnp.dot(h_ref[...], wr_ref[...],
                   preferred_element_type=jnp.float32) + br_ref[...])
    mu = jnp.mean(y, axis=1, keepdims=True)
    yc = y - mu
    var = jnp.mean(yc * yc, axis=1, keepdims=True)
    ln = yc * lax.rsqrt(var + 1e-5) * g_ref[...] + be_ref[...]
    o_ref[...] = jnp.maximum(ln, 0.0)


def _k2(agg, h, wr, dinv, bc, br, g, be):
    vec = lambda v: v.reshape(1, _HID)
    return pl.pallas_call(
        _k2_body,
        grid=(_N // _MB,),
        in_specs=[
            pl.BlockSpec((_NCHUNK, _MB, _CW), lambda i: (0, i, 0)),
            pl.BlockSpec((_MB, _HID), lambda i: (i, 0)),
            pl.BlockSpec((_HID, _HID), lambda i: (0, 0)),
            pl.BlockSpec((_MB, 1), lambda i: (i, 0)),
            pl.BlockSpec((1, _HID), lambda i: (0, 0)),
            pl.BlockSpec((1, _HID), lambda i: (0, 0)),
            pl.BlockSpec((1, _HID), lambda i: (0, 0)),
            pl.BlockSpec((1, _HID), lambda i: (0, 0)),
        ],
        out_specs=pl.BlockSpec((_MB, _HID), lambda i: (i, 0)),
        out_shape=jax.ShapeDtypeStruct((_N, _HID), jnp.float32),
    )(agg, h, wr, dinv, vec(bc), vec(br), vec(g), vec(be))


def kernel(x, edge_index, params):
    src = edge_index[0].astype(jnp.int32)
    dst = edge_index[1].astype(jnp.int32)
    npad = _EPAD - _E
    src_p = jnp.concatenate([src, jnp.zeros((npad,), jnp.int32)])
    dst_p = jnp.concatenate([dst, jnp.full((npad,), _N, jnp.int32)])

    words_p = jnp.bitwise_or(src_p, jnp.left_shift(dst_p, 16))

    ones_rows = jnp.ones((_B, _CW), jnp.float32)
    zeros_rows = jnp.zeros((_NR, _CW), jnp.float32)
    deg = _sc_deg(dst_p, ones_rows, zeros_rows)
    dinv = _dinv(deg)

    h = _matmul_bias(x, params['W_in'], params['b_in'])
    for l in range(4):
        xs, xsb = _k1(h, params[f'Wc{l}'], dinv)
        agg = _sc_agg(xs.reshape(_NCHUNK * _NR, _CW),
                      xsb.reshape(_NCHUNK * _NR, _CW // 2), words_p)
        h = _k2(agg, h, params[f'Wr{l}'], dinv,
                params[f'bc{l}'], params[f'br{l}'],
                params[f'g{l}'], params[f'be{l}'])
    return _matmul_bias(h, params['W_out'], params['b_out'])
